# baseline (speedup 1.0000x reference)
import functools

import jax
import jax.numpy as jnp
from jax import lax
from jax.experimental import pallas as pl
from jax.experimental.pallas import tpu as pltpu
from jax.experimental.pallas import tpu_sc as plsc

BLOCK = 64
NSEL = 16
LANES = 16


def _topk_sc(compression_scores, batch, num_blocks):
  nchunks = num_blocks // LANES
  info = plsc.get_sparse_core_info()
  nc, ns = info.num_cores, info.num_subcores
  assert nc * ns == batch, (nc, ns, batch)
  mesh = plsc.VectorSubcoreMesh(core_axis_name="c", subcore_axis_name="s")

  @functools.partial(
      pl.kernel,
      out_type=jax.ShapeDtypeStruct((batch, NSEL), jnp.int32),
      mesh=mesh,
      scratch_types=[
          pltpu.VMEM((num_blocks,), jnp.float32),
          pltpu.VMEM((NSEL,), jnp.int32),
      ],
  )
  def run(scores_hbm, idx_hbm, scores_v, idx_v):
    b = lax.axis_index("s") * nc + lax.axis_index("c")
    pltpu.sync_copy(scores_hbm.at[b], scores_v)

    chunks = [scores_v[pl.ds(LANES * i, LANES)] for i in range(nchunks)]
    gidx = [lax.iota(jnp.int32, LANES) + LANES * i for i in range(nchunks)]
    valid = [jnp.ones((LANES,), jnp.bool_) for _ in range(nchunks)]

    neg_inf = jnp.float32(-jnp.inf)
    big = jnp.int32(num_blocks)
    lane = lax.iota(jnp.int32, LANES)
    perms = [lane ^ s for s in (8, 4, 2, 1)]

    def butterfly(v, op):
      for s in range(4):
        v = op(v, v.at[perms[s]].get(mode="promise_in_bounds"))
      return v

    acc = jnp.zeros((LANES,), jnp.int32)
    for j in range(NSEL):
      masked = [jnp.where(valid[i], chunks[i], neg_inf) for i in range(nchunks)]
      mv = masked[0]
      for i in range(1, nchunks):
        mv = jnp.maximum(mv, masked[i])
      m = butterfly(mv, jnp.maximum)
      iv = jnp.where(valid[0] & (chunks[0] == m), gidx[0], big)
      for i in range(1, nchunks):
        iv = jnp.minimum(iv, jnp.where(valid[i] & (chunks[i] == m), gidx[i],
                                       big))
      sel_v = butterfly(iv, jnp.minimum)
      valid = [valid[i] & (gidx[i] != sel_v) for i in range(nchunks)]
      acc = jnp.where(lane == j, sel_v, acc)

    idx_v[...] = acc
    pltpu.sync_copy(idx_v, idx_hbm.at[b])

  return run(compression_scores)


def _gather_tc(keys, top_blocks, batch, key_dim):
  def body(idx_ref, x_ref, o_ref):
    del idx_ref
    o_ref[...] = x_ref[...]

  grid_spec = pltpu.PrefetchScalarGridSpec(
      num_scalar_prefetch=1,
      grid=(batch, NSEL),
      in_specs=[
          pl.BlockSpec((1, BLOCK, key_dim),
                       lambda b, j, idx: (b, idx[b, j], 0)),
      ],
      out_specs=pl.BlockSpec((1, BLOCK, key_dim),
                             lambda b, j, idx: (b, j, 0)),
  )
  return pl.pallas_call(
      body,
      grid_spec=grid_spec,
      out_shape=jax.ShapeDtypeStruct((batch, NSEL * BLOCK, key_dim),
                                     jnp.float32),
  )(top_blocks, keys)


def kernel(keys, compression_scores):
  batch, seq_len, key_dim = keys.shape
  num_blocks = seq_len // BLOCK
  top_blocks = _topk_sc(compression_scores, batch, num_blocks)
  return _gather_tc(keys, top_blocks, batch, key_dim)


# speedup vs baseline: 8.1487x; 8.1487x over previous
import functools

import jax
import jax.numpy as jnp
from jax import lax
from jax.experimental import pallas as pl
from jax.experimental.pallas import tpu as pltpu
from jax.experimental.pallas import tpu_sc as plsc

BLOCK = 64
NSEL = 16
LANES = 16


def kernel(keys, compression_scores):
  batch, seq_len, key_dim = keys.shape
  num_blocks = seq_len // BLOCK
  nchunks = num_blocks // LANES
  out_rows = NSEL * BLOCK
  nring = 12
  lag = 4

  info = plsc.get_sparse_core_info()
  nc, ns = info.num_cores, info.num_subcores
  assert nc * ns == batch, (nc, ns, batch)

  table = keys.reshape(batch * seq_len, key_dim)

  mesh = plsc.VectorSubcoreMesh(core_axis_name="c", subcore_axis_name="s")

  @functools.partial(
      pl.kernel,
      out_type=jax.ShapeDtypeStruct((batch * out_rows, key_dim), jnp.float32),
      mesh=mesh,
      scratch_types=[
          pltpu.VMEM((num_blocks,), jnp.float32),
          pltpu.VMEM((nring, BLOCK, key_dim), jnp.float32),
          pltpu.SemaphoreType.DMA,
          pltpu.SemaphoreType.DMA,
      ],
  )
  def run(table_hbm, scores_hbm, out_hbm, scores_v, buf, gsem, osem):
    b = lax.axis_index("s") * nc + lax.axis_index("c")
    pltpu.sync_copy(scores_hbm.at[b], scores_v)

    chunks = [scores_v[pl.ds(LANES * i, LANES)] for i in range(nchunks)]
    gidx = [lax.iota(jnp.int32, LANES) + LANES * i for i in range(nchunks)]
    valid = [jnp.ones((LANES,), jnp.bool_) for _ in range(nchunks)]

    neg_inf = jnp.float32(-jnp.inf)
    big = jnp.int32(num_blocks)
    lane = lax.iota(jnp.int32, LANES)
    perms = [lane ^ s for s in (8, 4, 2, 1)]

    def butterfly(v, op):
      for s in range(4):
        v = op(v, v.at[perms[s]].get(mode="promise_in_bounds"))
      return v

    seq_base = b * (num_blocks * BLOCK)
    out_base = b * out_rows
    gathers = [None] * NSEL
    outs = [None] * NSEL

    def start_out(j):
      gathers[j].wait()
      outs[j] = pltpu.async_copy(
          buf.at[j % nring],
          out_hbm.at[pl.ds(out_base + j * BLOCK, BLOCK)], osem)

    for j in range(NSEL):
      masked = [jnp.where(valid[i], chunks[i], neg_inf) for i in range(nchunks)]
      mv = masked[0]
      for i in range(1, nchunks):
        mv = jnp.maximum(mv, masked[i])
      m = butterfly(mv, jnp.maximum)
      iv = jnp.where(valid[0] & (chunks[0] == m), gidx[0], big)
      for i in range(1, nchunks):
        iv = jnp.minimum(iv, jnp.where(valid[i] & (chunks[i] == m), gidx[i],
                                       big))
      sel_v = butterfly(iv, jnp.minimum)
      valid = [valid[i] & (gidx[i] != sel_v) for i in range(nchunks)]
      sel = sel_v[0]
      if j >= nring:
        outs[j - nring].wait()
      gathers[j] = pltpu.async_copy(
          table_hbm.at[pl.ds(seq_base + sel * BLOCK, BLOCK)],
          buf.at[j % nring], gsem)
      if j >= lag:
        start_out(j - lag)
    for j in range(NSEL - lag, NSEL):
      start_out(j)
    for j in range(NSEL - nring, NSEL):
      outs[j].wait()

  out = run(table, compression_scores)
  return out.reshape(batch, out_rows, key_dim)
